# baseline (speedup 1.0000x reference)
import functools

import jax
import jax.numpy as jnp
from jax import lax
from jax.experimental import pallas as pl
from jax.experimental.pallas import tpu as pltpu
from jax.experimental.pallas import tpu_sc as plsc

N_NODES = 10000
N_EDGES = 320000
D_NODE = 128
D_EDGE = 16
D_OUT = 128
D_ATT = D_NODE + D_EDGE
D_SRC = 384
D_DST = 384
D_ACC = 128
NSUB = 16
CHUNK = 16
EPW = N_EDGES // NSUB
N_ACC = 10240
ROWS_PER_SUB = N_ACC // NSUB

NODE_BLK = 400
EDGE_BLK = 4000
D_ES = 80
BLK_E = 800
CPB = BLK_E // CHUNK
NBLK = EPW // BLK_E


def _node_pre_body(u_ref, v_ref, wu_ref, bu_ref, wv_ref, bv_ref,
                   wau_ref, bau_ref, wav_ref, bav_ref,
                   wvu_ref, bvu_ref, wvv_ref, bvv_ref,
                   wwu1_ref, wwv1_ref,
                   tsb_ref, tdb_ref, tsf_ref, tdf_ref, up_ref, vp_ref):
  u = u_ref[...]
  v = v_ref[...]
  he_u = jnp.dot(u, wu_ref[...], preferred_element_type=jnp.float32) + bu_ref[...]
  he_v = jnp.dot(v, wv_ref[...], preferred_element_type=jnp.float32) + bv_ref[...]
  att_u = jnp.dot(u, wau_ref[...], preferred_element_type=jnp.float32) + bau_ref[...]
  att_v = jnp.dot(v, wav_ref[...], preferred_element_type=jnp.float32) + bav_ref[...]
  tsb_ref[:, 0:D_NODE] = he_v
  tsb_ref[:, D_NODE:2 * D_NODE] = v
  tsb_ref[:, 2 * D_NODE:2 * D_NODE + 64] = jnp.dot(
      v, wwu1_ref[...], preferred_element_type=jnp.float32)
  tdb_ref[:, 0:D_NODE] = he_u
  tdb_ref[:, D_NODE:D_NODE + D_ATT] = att_u
  tsf_ref[:, 0:D_NODE] = he_u
  tsf_ref[:, D_NODE:2 * D_NODE] = u
  tsf_ref[:, 2 * D_NODE:2 * D_NODE + 64] = jnp.dot(
      u, wwv1_ref[...], preferred_element_type=jnp.float32)
  tdf_ref[:, 0:D_NODE] = he_v
  tdf_ref[:, D_NODE:D_NODE + D_ATT] = att_v
  up_ref[...] = jnp.dot(u, wvu_ref[...], preferred_element_type=jnp.float32) + bvu_ref[...]
  vp_ref[...] = jnp.dot(v, wvv_ref[...], preferred_element_type=jnp.float32) + bvv_ref[...]


def _full(shape):
  return pl.BlockSpec(shape, lambda i: (0,) * len(shape))


def _node_pre(u_feat, v_feat, p):
  nblk = N_NODES // NODE_BLK
  row = lambda w: pl.BlockSpec((NODE_BLK, w), lambda i: (i, 0))
  f32 = jnp.float32
  outs = [
      jax.ShapeDtypeStruct((N_NODES, D_SRC), f32),
      jax.ShapeDtypeStruct((N_NODES, D_DST), f32),
      jax.ShapeDtypeStruct((N_NODES, D_SRC), f32),
      jax.ShapeDtypeStruct((N_NODES, D_DST), f32),
      jax.ShapeDtypeStruct((N_NODES, 64), f32),
      jax.ShapeDtypeStruct((N_NODES, 64), f32),
  ]
  return pl.pallas_call(
      _node_pre_body,
      grid=(nblk,),
      in_specs=[
          row(D_NODE), row(D_NODE),
          _full((D_NODE, D_OUT)), _full((1, D_OUT)),
          _full((D_NODE, D_OUT)), _full((1, D_OUT)),
          _full((D_NODE, D_ATT)), _full((1, D_ATT)),
          _full((D_NODE, D_ATT)), _full((1, D_ATT)),
          _full((D_NODE, 64)), _full((1, 64)),
          _full((D_NODE, 64)), _full((1, 64)),
          _full((D_NODE, 64)), _full((D_NODE, 64)),
      ],
      out_specs=[row(D_SRC), row(D_DST), row(D_SRC), row(D_DST),
                 row(64), row(64)],
      out_shape=outs,
  )(u_feat, v_feat,
    p['Wu'], p['bu'].reshape(1, -1), p['Wv'], p['bv'].reshape(1, -1),
    p['Wau'], p['bau'].reshape(1, -1), p['Wav'], p['bav'].reshape(1, -1),
    p['Wvu'], p['bvu'].reshape(1, -1), p['Wvv'], p['bvv'].reshape(1, -1),
    p['Wwu'][:D_NODE], p['Wwv'][:D_NODE])


def _edge_pre_body(f_ref, b_ref, wwu2_ref, wwv2_ref, esf_ref, esb_ref):
  f = f_ref[...]
  b = b_ref[...]
  esf_ref[:, 0:64] = jnp.dot(f, wwv2_ref[...],
                             preferred_element_type=jnp.float32)
  esb_ref[:, 0:64] = jnp.dot(b, wwu2_ref[...],
                             preferred_element_type=jnp.float32)
  esf_ref[:, 64:D_ES] = f
  esb_ref[:, 64:D_ES] = b


def _edge_pre(f_feat, b_feat, wwu2, wwv2):
  nblk = N_EDGES // EDGE_BLK
  row = lambda w: pl.BlockSpec((EDGE_BLK, w), lambda i: (i, 0))
  f32 = jnp.float32
  return pl.pallas_call(
      _edge_pre_body,
      grid=(nblk,),
      in_specs=[row(D_EDGE), row(D_EDGE), _full((D_EDGE, 64)),
                _full((D_EDGE, 64))],
      out_specs=[row(D_ES), row(D_ES)],
      out_shape=[jax.ShapeDtypeStruct((N_EDGES, D_ES), f32),
                 jax.ShapeDtypeStruct((N_EDGES, D_ES), f32)],
  )(f_feat, b_feat, wwu2, wwv2)


def _edge_post_body(hpf_ref, hpb_ref, f_ref, b_ref, we_ref, be_ref,
                    hf_ref, hb_ref):
  we = we_ref[...]
  be = be_ref[...]
  hf_ref[...] = hpf_ref[...] + (
      jnp.dot(f_ref[...], we, preferred_element_type=jnp.float32) + be)
  hb_ref[...] = hpb_ref[...] + (
      jnp.dot(b_ref[...], we, preferred_element_type=jnp.float32) + be)


def _edge_post(hpf, hpb, f_feat, b_feat, we, be):
  nblk = N_EDGES // EDGE_BLK
  row = lambda w: pl.BlockSpec((EDGE_BLK, w), lambda i: (i, 0))
  f32 = jnp.float32
  return pl.pallas_call(
      _edge_post_body,
      grid=(nblk,),
      in_specs=[row(D_OUT), row(D_OUT), row(D_EDGE), row(D_EDGE),
                _full((D_EDGE, D_OUT)), _full((1, D_OUT))],
      out_specs=[row(D_OUT), row(D_OUT)],
      out_shape=[jax.ShapeDtypeStruct((N_EDGES, D_OUT), f32),
                 jax.ShapeDtypeStruct((N_EDGES, D_OUT), f32)],
  )(hpf, hpb, f_feat, b_feat, we, be.reshape(1, -1))


_GATHER_DN = lax.GatherDimensionNumbers(
    offset_dims=(), collapsed_slice_dims=(0,), start_index_map=(0,))


def _lane_perm(x, ix):
  return lax.gather(x, ix[:, None], _GATHER_DN, slice_sizes=(1,),
                    mode=lax.GatherScatterMode.PROMISE_IN_BOUNDS)


def _sc_body(src_b, dst_b, src_f, dst_f, tsb, tdb, tsf, tdf,
             esb, esf, zeros_h,
             hb_o, hf_o, accu_o, accv_o,
             sblk, dblk,
             sr0, sr1, dr0, dr1, ev0, ev1, ov0, ov1, av0, av1,
             ss0, ss1, sd0, sd1, se0, se1, so0, so1, sa0, sa1, acc_sh):
  d = lax.axis_index("c")
  sid = lax.axis_index("s")

  pltpu.sync_copy(zeros_h.at[pl.ds(sid * ROWS_PER_SUB, ROWS_PER_SUB)],
                  acc_sh.at[pl.ds(sid * ROWS_PER_SUB, ROWS_PER_SUB)])
  plsc.subcore_barrier()

  lanes = lax.iota(jnp.int32, 16)
  lane0 = lanes == 0
  zero16 = jnp.zeros((16,), jnp.float32)
  perms = [lanes ^ k for k in (8, 4, 2, 1)]
  SET0 = (sr0, dr0, ev0, ov0, av0, ss0, sd0, se0, so0, sa0)
  SET1 = (sr1, dr1, ev1, ov1, av1, ss1, sd1, se1, so1, sa1)

  def run_direction(src_i, dst_i, tsrc, tdst, es, h_o, acc_o):
    def svec(c):
      return sblk[pl.ds(c * CHUNK, CHUNK)]

    def dvec(c):
      return dblk[pl.ds(c * CHUNK, CHUNK)]

    def prefetch(c, st, gbase):
      sr, dr, ev, _, _, ss, sd, se, _, _ = st
      pltpu.async_copy(tsrc.at[svec(c)], sr, ss)
      pltpu.async_copy(tdst.at[dvec(c)], dr, sd)
      pltpu.async_copy(es.at[pl.ds(gbase + c * CHUNK, CHUNK)], ev, se)

    def wait_in(st):
      sr, dr, ev, _, _, ss, sd, se, _, _ = st
      pltpu.make_async_copy(tsrc.at[pl.ds(0, CHUNK)], sr, ss).wait()
      pltpu.make_async_copy(tdst.at[pl.ds(0, CHUNK)], dr, sd).wait()
      pltpu.make_async_copy(es.at[pl.ds(0, CHUNK)], ev, se).wait()

    def wait_out(st):
      _, _, _, ov, av, _, _, _, so, sa = st
      pltpu.make_async_copy(ov, h_o.at[pl.ds(0, CHUNK)], so).wait()
      pltpu.make_async_copy(av, acc_sh.at[pl.ds(0, CHUNK)], sa).wait()

    def compute(st):
      sr, dr, ev, ov, av, _, _, _, _, _ = st

      def edge_pair(e2, carry):
        for sub in range(4):
          e = e2 * 4 + sub
          acc = sr[e, pl.ds(D_NODE, 16)] * dr[e, pl.ds(D_NODE, 16)]
          for k in range(1, 8):
            acc = acc + (sr[e, pl.ds(D_NODE + k * 16, 16)] *
                         dr[e, pl.ds(D_NODE + k * 16, 16)])
          acc = acc + (ev[e, pl.ds(64, 16)] *
                       dr[e, pl.ds(2 * D_NODE, 16)])
          for ix in perms:
            acc = acc + _lane_perm(acc, ix)
          exv = jnp.exp(jnp.minimum(acc, 70.0))
          for k in range(8):
            off = k * 16
            ov[e, pl.ds(off, 16)] = (sr[e, pl.ds(off, 16)] +
                                     dr[e, pl.ds(off, 16)])
          for k in range(4):
            off = k * 16
            av[e, pl.ds(off, 16)] = (sr[e, pl.ds(2 * D_NODE + off, 16)] +
                                     ev[e, pl.ds(off, 16)]) * exv
          av[e, pl.ds(64, 16)] = jnp.where(lane0, exv, 0.0)
          av[e, pl.ds(80, 16)] = zero16
          av[e, pl.ds(96, 16)] = zero16
          av[e, pl.ds(112, 16)] = zero16
        return carry

      lax.fori_loop(0, CHUNK // 4, edge_pair, 0)

    def outs(c, st, gbase):
      _, _, _, ov, av, _, _, _, so, sa = st
      pltpu.async_copy(ov, h_o.at[pl.ds(gbase + c * CHUNK, CHUNK)], so)
      pltpu.async_copy(av, acc_sh.at[dvec(c)], sa, add=True)

    def block_body(bi, carry):
      gbase = sid * EPW + bi * BLK_E
      pltpu.sync_copy(src_i.at[pl.ds(gbase, BLK_E)], sblk)
      pltpu.sync_copy(dst_i.at[pl.ds(gbase, BLK_E)], dblk)
      prefetch(0, SET0, gbase)
      wait_in(SET0)
      prefetch(1, SET1, gbase)
      compute(SET0)
      outs(0, SET0, gbase)
      wait_in(SET1)
      prefetch(2, SET0, gbase)
      compute(SET1)
      outs(1, SET1, gbase)

      def pair(j, c2):
        cA = 2 * j
        wait_in(SET0)
        prefetch(cA + 1, SET1, gbase)
        wait_out(SET0)
        compute(SET0)
        outs(cA, SET0, gbase)
        wait_in(SET1)
        prefetch(cA + 2, SET0, gbase)
        wait_out(SET1)
        compute(SET1)
        outs(cA + 1, SET1, gbase)
        return c2

      lax.fori_loop(1, CPB // 2 - 1, pair, 0)
      wait_in(SET0)
      prefetch(CPB - 1, SET1, gbase)
      wait_out(SET0)
      compute(SET0)
      outs(CPB - 2, SET0, gbase)
      wait_in(SET1)
      wait_out(SET1)
      compute(SET1)
      outs(CPB - 1, SET1, gbase)
      wait_out(SET0)
      wait_out(SET1)
      return carry

    lax.fori_loop(0, NBLK, block_body, 0)
    plsc.subcore_barrier()
    pltpu.sync_copy(acc_sh.at[pl.ds(sid * ROWS_PER_SUB, ROWS_PER_SUB)],
                    acc_o.at[pl.ds(sid * ROWS_PER_SUB, ROWS_PER_SUB)])

  @pl.when(d == 0)
  def _():
    run_direction(src_b, dst_b, tsb, tdb, esb, hb_o, accu_o)

  @pl.when(d == 1)
  def _():
    run_direction(src_f, dst_f, tsf, tdf, esf, hf_o, accv_o)


def _sc_edges(src_b, dst_b, src_f, dst_f, tsb, tdb, tsf, tdf,
              esb, esf, zeros_h):
  f32 = jnp.float32
  mesh = plsc.VectorSubcoreMesh(core_axis_name="c", subcore_axis_name="s")
  buf = lambda w: pltpu.VMEM((CHUNK, w), f32)
  fn = pl.kernel(
      _sc_body,
      out_type=[
          jax.ShapeDtypeStruct((N_EDGES, D_OUT), f32),
          jax.ShapeDtypeStruct((N_EDGES, D_OUT), f32),
          jax.ShapeDtypeStruct((N_ACC, D_ACC), f32),
          jax.ShapeDtypeStruct((N_ACC, D_ACC), f32),
      ],
      mesh=mesh,
      scratch_types=[
          pltpu.VMEM((BLK_E,), jnp.int32),
          pltpu.VMEM((BLK_E,), jnp.int32),
          buf(D_SRC), buf(D_SRC), buf(D_DST), buf(D_DST),
          buf(D_ES), buf(D_ES), buf(D_OUT), buf(D_OUT),
          buf(D_ACC), buf(D_ACC),
          pltpu.SemaphoreType.DMA, pltpu.SemaphoreType.DMA,
          pltpu.SemaphoreType.DMA, pltpu.SemaphoreType.DMA,
          pltpu.SemaphoreType.DMA, pltpu.SemaphoreType.DMA,
          pltpu.SemaphoreType.DMA, pltpu.SemaphoreType.DMA,
          pltpu.SemaphoreType.DMA, pltpu.SemaphoreType.DMA,
          pltpu.VMEM_SHARED((N_ACC, D_ACC), f32),
      ],
  )
  return fn(src_b, dst_b, src_f, dst_f, tsb, tdb, tsf, tdf,
            esb, esf, zeros_h)


def _final_body(accu_ref, accv_ref, up_ref, vp_ref,
                bwu_ref, bwv_ref, hu_ref, hv_ref):
  accu = accu_ref[...]
  accv = accv_ref[...]
  h_nu = accu[:, 0:64] / jnp.maximum(accu[:, 64:65], 1e-38) + bwu_ref[...]
  h_nv = accv[:, 0:64] / jnp.maximum(accv[:, 64:65], 1e-38) + bwv_ref[...]
  hu_ref[:, 0:64] = up_ref[...]
  hu_ref[:, 64:D_OUT] = h_nu
  hv_ref[:, 0:64] = vp_ref[...]
  hv_ref[:, 64:D_OUT] = h_nv


def _finalize(accu, accv, upart, vpart, p):
  nblk = N_NODES // NODE_BLK
  row = lambda w: pl.BlockSpec((NODE_BLK, w), lambda i: (i, 0))
  f32 = jnp.float32
  return pl.pallas_call(
      _final_body,
      grid=(nblk,),
      in_specs=[row(D_ACC), row(D_ACC), row(64), row(64),
                _full((1, 64)), _full((1, 64))],
      out_specs=[row(D_OUT), row(D_OUT)],
      out_shape=[jax.ShapeDtypeStruct((N_NODES, D_OUT), f32),
                 jax.ShapeDtypeStruct((N_NODES, D_OUT), f32)],
  )(accu, accv, upart, vpart,
    p['bwu'].reshape(1, -1), p['bwv'].reshape(1, -1))


@jax.jit
def kernel(f_feat, b_feat, u_feat, v_feat, edge_index_f, edge_index_b, params):
  src_f = edge_index_f[0].astype(jnp.int32)
  dst_f = edge_index_f[1].astype(jnp.int32)
  src_b = edge_index_b[0].astype(jnp.int32)
  dst_b = edge_index_b[1].astype(jnp.int32)

  tsb, tdb, tsf, tdf, upart, vpart = _node_pre(u_feat, v_feat, params)
  esf, esb = _edge_pre(f_feat, b_feat,
                       params['Wwu'][D_NODE:], params['Wwv'][D_NODE:])
  zeros_h = jnp.zeros((N_ACC, D_ACC), jnp.float32)

  hbp, hfp, accu, accv = _sc_edges(src_b, dst_b, src_f, dst_f,
                                   tsb, tdb, tsf, tdf, esb, esf, zeros_h)
  hf, hb = _edge_post(hfp, hbp, f_feat, b_feat, params['We'], params['be'])
  hu, hv = _finalize(accu[:N_NODES], accv[:N_NODES], upart, vpart, params)
  return (hf, hb, hu, hv)


# speedup vs baseline: 1.0181x; 1.0181x over previous
import functools

import jax
import jax.numpy as jnp
from jax import lax
from jax.experimental import pallas as pl
from jax.experimental.pallas import tpu as pltpu
from jax.experimental.pallas import tpu_sc as plsc

N_NODES = 10000
N_EDGES = 320000
D_NODE = 128
D_EDGE = 16
D_OUT = 128
D_ATT = D_NODE + D_EDGE
D_SRC = 384
D_DST = 384
D_ACC = 128
NSUB = 16
CHUNK = 16
EPW = N_EDGES // NSUB
N_ACC = 10240
ROWS_PER_SUB = N_ACC // NSUB

NODE_BLK = 400
EDGE_BLK = 4000
D_ES = 208
BLK_E = 800
CPB = BLK_E // CHUNK
NBLK = EPW // BLK_E


def _node_pre_body(u_ref, v_ref, wu_ref, bu_ref, wv_ref, bv_ref,
                   wau_ref, bau_ref, wav_ref, bav_ref,
                   wvu_ref, bvu_ref, wvv_ref, bvv_ref,
                   wwu1_ref, wwv1_ref,
                   tsb_ref, tdb_ref, tsf_ref, tdf_ref, up_ref, vp_ref):
  u = u_ref[...]
  v = v_ref[...]
  he_u = jnp.dot(u, wu_ref[...], preferred_element_type=jnp.float32) + bu_ref[...]
  he_v = jnp.dot(v, wv_ref[...], preferred_element_type=jnp.float32) + bv_ref[...]
  att_u = jnp.dot(u, wau_ref[...], preferred_element_type=jnp.float32) + bau_ref[...]
  att_v = jnp.dot(v, wav_ref[...], preferred_element_type=jnp.float32) + bav_ref[...]
  tsb_ref[:, 0:D_NODE] = he_v
  tsb_ref[:, D_NODE:2 * D_NODE] = v
  tsb_ref[:, 2 * D_NODE:2 * D_NODE + 64] = jnp.dot(
      v, wwu1_ref[...], preferred_element_type=jnp.float32)
  tdb_ref[:, 0:D_NODE] = he_u
  tdb_ref[:, D_NODE:D_NODE + D_ATT] = att_u
  tsf_ref[:, 0:D_NODE] = he_u
  tsf_ref[:, D_NODE:2 * D_NODE] = u
  tsf_ref[:, 2 * D_NODE:2 * D_NODE + 64] = jnp.dot(
      u, wwv1_ref[...], preferred_element_type=jnp.float32)
  tdf_ref[:, 0:D_NODE] = he_v
  tdf_ref[:, D_NODE:D_NODE + D_ATT] = att_v
  up_ref[...] = jnp.dot(u, wvu_ref[...], preferred_element_type=jnp.float32) + bvu_ref[...]
  vp_ref[...] = jnp.dot(v, wvv_ref[...], preferred_element_type=jnp.float32) + bvv_ref[...]


def _full(shape):
  return pl.BlockSpec(shape, lambda i: (0,) * len(shape))


def _node_pre(u_feat, v_feat, p):
  nblk = N_NODES // NODE_BLK
  row = lambda w: pl.BlockSpec((NODE_BLK, w), lambda i: (i, 0))
  f32 = jnp.float32
  outs = [
      jax.ShapeDtypeStruct((N_NODES, D_SRC), f32),
      jax.ShapeDtypeStruct((N_NODES, D_DST), f32),
      jax.ShapeDtypeStruct((N_NODES, D_SRC), f32),
      jax.ShapeDtypeStruct((N_NODES, D_DST), f32),
      jax.ShapeDtypeStruct((N_NODES, 64), f32),
      jax.ShapeDtypeStruct((N_NODES, 64), f32),
  ]
  return pl.pallas_call(
      _node_pre_body,
      grid=(nblk,),
      in_specs=[
          row(D_NODE), row(D_NODE),
          _full((D_NODE, D_OUT)), _full((1, D_OUT)),
          _full((D_NODE, D_OUT)), _full((1, D_OUT)),
          _full((D_NODE, D_ATT)), _full((1, D_ATT)),
          _full((D_NODE, D_ATT)), _full((1, D_ATT)),
          _full((D_NODE, 64)), _full((1, 64)),
          _full((D_NODE, 64)), _full((1, 64)),
          _full((D_NODE, 64)), _full((D_NODE, 64)),
      ],
      out_specs=[row(D_SRC), row(D_DST), row(D_SRC), row(D_DST),
                 row(64), row(64)],
      out_shape=outs,
  )(u_feat, v_feat,
    p['Wu'], p['bu'].reshape(1, -1), p['Wv'], p['bv'].reshape(1, -1),
    p['Wau'], p['bau'].reshape(1, -1), p['Wav'], p['bav'].reshape(1, -1),
    p['Wvu'], p['bvu'].reshape(1, -1), p['Wvv'], p['bvv'].reshape(1, -1),
    p['Wwu'][:D_NODE], p['Wwv'][:D_NODE])


def _edge_pre_body(f_ref, b_ref, we_ref, be_ref, wwu2_ref, wwv2_ref,
                   esf_ref, esb_ref):
  f = f_ref[...]
  b = b_ref[...]
  we = we_ref[...]
  be = be_ref[...]
  esf_ref[:, 0:D_OUT] = jnp.dot(f, we, preferred_element_type=jnp.float32) + be
  esb_ref[:, 0:D_OUT] = jnp.dot(b, we, preferred_element_type=jnp.float32) + be
  esf_ref[:, D_OUT:D_OUT + 64] = jnp.dot(f, wwv2_ref[...],
                                         preferred_element_type=jnp.float32)
  esb_ref[:, D_OUT:D_OUT + 64] = jnp.dot(b, wwu2_ref[...],
                                         preferred_element_type=jnp.float32)
  esf_ref[:, D_OUT + 64:D_ES] = f
  esb_ref[:, D_OUT + 64:D_ES] = b


def _edge_pre(f_feat, b_feat, we, be, wwu2, wwv2):
  nblk = N_EDGES // EDGE_BLK
  row = lambda w: pl.BlockSpec((EDGE_BLK, w), lambda i: (i, 0))
  f32 = jnp.float32
  return pl.pallas_call(
      _edge_pre_body,
      grid=(nblk,),
      in_specs=[row(D_EDGE), row(D_EDGE), _full((D_EDGE, D_OUT)),
                _full((1, D_OUT)), _full((D_EDGE, 64)), _full((D_EDGE, 64))],
      out_specs=[row(D_ES), row(D_ES)],
      out_shape=[jax.ShapeDtypeStruct((N_EDGES, D_ES), f32),
                 jax.ShapeDtypeStruct((N_EDGES, D_ES), f32)],
  )(f_feat, b_feat, we, be.reshape(1, -1), wwu2, wwv2)


_GATHER_DN = lax.GatherDimensionNumbers(
    offset_dims=(), collapsed_slice_dims=(0,), start_index_map=(0,))


def _lane_perm(x, ix):
  return lax.gather(x, ix[:, None], _GATHER_DN, slice_sizes=(1,),
                    mode=lax.GatherScatterMode.PROMISE_IN_BOUNDS)


def _sc_body(src_b, dst_b, src_f, dst_f, tsb, tdb, tsf, tdf,
             esb, esf, zeros_h,
             hb_o, hf_o, accu_o, accv_o,
             sblk, dblk,
             sr0, sr1, dr0, dr1, ev0, ev1, ov0, ov1, av0, av1,
             ss0, ss1, sd0, sd1, se0, se1, so0, so1, sa0, sa1, acc_sh):
  d = lax.axis_index("c")
  sid = lax.axis_index("s")

  pltpu.sync_copy(zeros_h.at[pl.ds(sid * ROWS_PER_SUB, ROWS_PER_SUB)],
                  acc_sh.at[pl.ds(sid * ROWS_PER_SUB, ROWS_PER_SUB)])
  plsc.subcore_barrier()

  lanes = lax.iota(jnp.int32, 16)
  lane0 = lanes == 0
  zero16 = jnp.zeros((16,), jnp.float32)
  perms = [lanes ^ k for k in (8, 4, 2, 1)]
  SET0 = (sr0, dr0, ev0, ov0, av0, ss0, sd0, se0, so0, sa0)
  SET1 = (sr1, dr1, ev1, ov1, av1, ss1, sd1, se1, so1, sa1)

  def run_direction(src_i, dst_i, tsrc, tdst, es, h_o, acc_o):
    def svec(c):
      return sblk[pl.ds(c * CHUNK, CHUNK)]

    def dvec(c):
      return dblk[pl.ds(c * CHUNK, CHUNK)]

    def prefetch(c, st, gbase):
      sr, dr, ev, _, _, ss, sd, se, _, _ = st
      pltpu.async_copy(tsrc.at[svec(c)], sr, ss)
      pltpu.async_copy(tdst.at[dvec(c)], dr, sd)
      pltpu.async_copy(es.at[pl.ds(gbase + c * CHUNK, CHUNK)], ev, se)

    def wait_in(st):
      sr, dr, ev, _, _, ss, sd, se, _, _ = st
      pltpu.make_async_copy(tsrc.at[pl.ds(0, CHUNK)], sr, ss).wait()
      pltpu.make_async_copy(tdst.at[pl.ds(0, CHUNK)], dr, sd).wait()
      pltpu.make_async_copy(es.at[pl.ds(0, CHUNK)], ev, se).wait()

    def wait_out(st):
      _, _, _, ov, av, _, _, _, so, sa = st
      pltpu.make_async_copy(ov, h_o.at[pl.ds(0, CHUNK)], so).wait()
      pltpu.make_async_copy(av, acc_sh.at[pl.ds(0, CHUNK)], sa).wait()

    def compute(st):
      sr, dr, ev, ov, av, _, _, _, _, _ = st

      def edge_pair(e2, carry):
        for sub in range(4):
          e = e2 * 4 + sub
          acc = sr[e, pl.ds(D_NODE, 16)] * dr[e, pl.ds(D_NODE, 16)]
          for k in range(1, 8):
            acc = acc + (sr[e, pl.ds(D_NODE + k * 16, 16)] *
                         dr[e, pl.ds(D_NODE + k * 16, 16)])
          acc = acc + (ev[e, pl.ds(D_OUT + 64, 16)] *
                       dr[e, pl.ds(2 * D_NODE, 16)])
          for ix in perms:
            acc = acc + _lane_perm(acc, ix)
          exv = jnp.exp(jnp.minimum(acc, 70.0))
          for k in range(8):
            off = k * 16
            ov[e, pl.ds(off, 16)] = (ev[e, pl.ds(off, 16)] +
                                     sr[e, pl.ds(off, 16)] +
                                     dr[e, pl.ds(off, 16)])
          for k in range(4):
            off = k * 16
            av[e, pl.ds(off, 16)] = (sr[e, pl.ds(2 * D_NODE + off, 16)] +
                                     ev[e, pl.ds(D_OUT + off, 16)]) * exv
          av[e, pl.ds(64, 16)] = jnp.where(lane0, exv, 0.0)
          av[e, pl.ds(80, 16)] = zero16
          av[e, pl.ds(96, 16)] = zero16
          av[e, pl.ds(112, 16)] = zero16
        return carry

      lax.fori_loop(0, CHUNK // 4, edge_pair, 0)

    def outs(c, st, gbase):
      _, _, _, ov, av, _, _, _, so, sa = st
      pltpu.async_copy(ov, h_o.at[pl.ds(gbase + c * CHUNK, CHUNK)], so)
      pltpu.async_copy(av, acc_sh.at[dvec(c)], sa, add=True)

    def block_body(bi, carry):
      gbase = sid * EPW + bi * BLK_E
      pltpu.sync_copy(src_i.at[pl.ds(gbase, BLK_E)], sblk)
      pltpu.sync_copy(dst_i.at[pl.ds(gbase, BLK_E)], dblk)
      prefetch(0, SET0, gbase)
      wait_in(SET0)
      prefetch(1, SET1, gbase)
      compute(SET0)
      outs(0, SET0, gbase)
      wait_in(SET1)
      prefetch(2, SET0, gbase)
      compute(SET1)
      outs(1, SET1, gbase)

      def pair(j, c2):
        cA = 2 * j
        wait_in(SET0)
        prefetch(cA + 1, SET1, gbase)
        wait_out(SET0)
        compute(SET0)
        outs(cA, SET0, gbase)
        wait_in(SET1)
        prefetch(cA + 2, SET0, gbase)
        wait_out(SET1)
        compute(SET1)
        outs(cA + 1, SET1, gbase)
        return c2

      lax.fori_loop(1, CPB // 2 - 1, pair, 0)
      wait_in(SET0)
      prefetch(CPB - 1, SET1, gbase)
      wait_out(SET0)
      compute(SET0)
      outs(CPB - 2, SET0, gbase)
      wait_in(SET1)
      wait_out(SET1)
      compute(SET1)
      outs(CPB - 1, SET1, gbase)
      wait_out(SET0)
      wait_out(SET1)
      return carry

    lax.fori_loop(0, NBLK, block_body, 0)
    plsc.subcore_barrier()
    pltpu.sync_copy(acc_sh.at[pl.ds(sid * ROWS_PER_SUB, ROWS_PER_SUB)],
                    acc_o.at[pl.ds(sid * ROWS_PER_SUB, ROWS_PER_SUB)])

  @pl.when(d == 0)
  def _():
    run_direction(src_b, dst_b, tsb, tdb, esb, hb_o, accu_o)

  @pl.when(d == 1)
  def _():
    run_direction(src_f, dst_f, tsf, tdf, esf, hf_o, accv_o)


def _sc_edges(src_b, dst_b, src_f, dst_f, tsb, tdb, tsf, tdf,
              esb, esf, zeros_h):
  f32 = jnp.float32
  mesh = plsc.VectorSubcoreMesh(core_axis_name="c", subcore_axis_name="s")
  buf = lambda w: pltpu.VMEM((CHUNK, w), f32)
  fn = pl.kernel(
      _sc_body,
      out_type=[
          jax.ShapeDtypeStruct((N_EDGES, D_OUT), f32),
          jax.ShapeDtypeStruct((N_EDGES, D_OUT), f32),
          jax.ShapeDtypeStruct((N_ACC, D_ACC), f32),
          jax.ShapeDtypeStruct((N_ACC, D_ACC), f32),
      ],
      mesh=mesh,
      scratch_types=[
          pltpu.VMEM((BLK_E,), jnp.int32),
          pltpu.VMEM((BLK_E,), jnp.int32),
          buf(D_SRC), buf(D_SRC), buf(D_DST), buf(D_DST),
          buf(D_ES), buf(D_ES), buf(D_OUT), buf(D_OUT),
          buf(D_ACC), buf(D_ACC),
          pltpu.SemaphoreType.DMA, pltpu.SemaphoreType.DMA,
          pltpu.SemaphoreType.DMA, pltpu.SemaphoreType.DMA,
          pltpu.SemaphoreType.DMA, pltpu.SemaphoreType.DMA,
          pltpu.SemaphoreType.DMA, pltpu.SemaphoreType.DMA,
          pltpu.SemaphoreType.DMA, pltpu.SemaphoreType.DMA,
          pltpu.VMEM_SHARED((N_ACC, D_ACC), f32),
      ],
  )
  return fn(src_b, dst_b, src_f, dst_f, tsb, tdb, tsf, tdf,
            esb, esf, zeros_h)


def _final_body(accu_ref, accv_ref, up_ref, vp_ref,
                bwu_ref, bwv_ref, hu_ref, hv_ref):
  accu = accu_ref[...]
  accv = accv_ref[...]
  h_nu = accu[:, 0:64] / jnp.maximum(accu[:, 64:65], 1e-38) + bwu_ref[...]
  h_nv = accv[:, 0:64] / jnp.maximum(accv[:, 64:65], 1e-38) + bwv_ref[...]
  hu_ref[:, 0:64] = up_ref[...]
  hu_ref[:, 64:D_OUT] = h_nu
  hv_ref[:, 0:64] = vp_ref[...]
  hv_ref[:, 64:D_OUT] = h_nv


def _finalize(accu, accv, upart, vpart, p):
  nblk = N_NODES // NODE_BLK
  row = lambda w: pl.BlockSpec((NODE_BLK, w), lambda i: (i, 0))
  f32 = jnp.float32
  return pl.pallas_call(
      _final_body,
      grid=(nblk,),
      in_specs=[row(D_ACC), row(D_ACC), row(64), row(64),
                _full((1, 64)), _full((1, 64))],
      out_specs=[row(D_OUT), row(D_OUT)],
      out_shape=[jax.ShapeDtypeStruct((N_NODES, D_OUT), f32),
                 jax.ShapeDtypeStruct((N_NODES, D_OUT), f32)],
  )(accu, accv, upart, vpart,
    p['bwu'].reshape(1, -1), p['bwv'].reshape(1, -1))


@jax.jit
def kernel(f_feat, b_feat, u_feat, v_feat, edge_index_f, edge_index_b, params):
  src_f = edge_index_f[0].astype(jnp.int32)
  dst_f = edge_index_f[1].astype(jnp.int32)
  src_b = edge_index_b[0].astype(jnp.int32)
  dst_b = edge_index_b[1].astype(jnp.int32)

  tsb, tdb, tsf, tdf, upart, vpart = _node_pre(u_feat, v_feat, params)
  esf, esb = _edge_pre(f_feat, b_feat, params['We'], params['be'],
                       params['Wwu'][D_NODE:], params['Wwv'][D_NODE:])
  zeros_h = jnp.zeros((N_ACC, D_ACC), jnp.float32)

  hb, hf, accu, accv = _sc_edges(src_b, dst_b, src_f, dst_f,
                                 tsb, tdb, tsf, tdf, esb, esf, zeros_h)
  hu, hv = _finalize(accu[:N_NODES], accv[:N_NODES], upart, vpart, params)
  return (hf, hb, hu, hv)
